# baseline (speedup 1.0000x reference)
import functools

import jax
import jax.numpy as jnp
from jax import lax
from jax.experimental import pallas as pl
from jax.experimental.pallas import tpu as pltpu
from jax.experimental.pallas import tpu_sc as plsc

N = 10000
IN_DIM = 128
HID = 16
HEADS = 8
OUT_DIM = 64
HD = HEADS * HID

NC = 2
NS = 16
NW = NC * NS
SUB = 64
NBUF = 2
IDXN = 4
E_RAW = 320000
NIT = -(-E_RAW // (NW * SUB))
NIT += NIT % 2
EPW = NIT * SUB
E_PAD = EPW * NW

NPAD = N + 16
ZSH = 632
C1 = HD + 16
C2 = 80
RB = 400

_f32 = jnp.float32
_i32 = jnp.int32

_GD = lax.GatherDimensionNumbers(offset_dims=(), collapsed_slice_dims=(0,),
                                 start_index_map=(0,))


def _dyn_gather(vec, idx):
    return lax.gather(vec, idx[:, None], _GD, (1,),
                      mode=lax.GatherScatterMode.PROMISE_IN_BOUNDS)


def _mm1_body(x_ref, wa_ref, wb_ref, fex_ref, er_ref):
    fex_ref[...] = jnp.dot(x_ref[...], wa_ref[...], preferred_element_type=_f32)
    er_ref[...] = jnp.dot(x_ref[...], wb_ref[...], preferred_element_type=_f32)


_mm1 = pl.pallas_call(
    _mm1_body,
    grid=(N // RB,),
    in_specs=[
        pl.BlockSpec((RB, IN_DIM), lambda i: (i, 0)),
        pl.BlockSpec((IN_DIM, C1), lambda i: (0, 0)),
        pl.BlockSpec((IN_DIM, 16), lambda i: (0, 0)),
    ],
    out_specs=[
        pl.BlockSpec((RB, C1), lambda i: (i, 0)),
        pl.BlockSpec((RB, 16), lambda i: (i, 0)),
    ],
    out_shape=[
        jax.ShapeDtypeStruct((NPAD, C1), _f32),
        jax.ShapeDtypeStruct((NPAD, 16), _f32),
    ],
)


def _mid_body(p0_ref, p1_ref, e8_ref, b1_ref, wa_ref, wb_ref,
              fex2_ref, er2_ref):
    acc = p0_ref[...] + p1_ref[...]
    s8 = acc[:, HD:HD + 8]
    sfull = jnp.dot(s8, e8_ref[...], preferred_element_type=_f32)
    h1 = jnp.maximum(acc[:, :HD] / (sfull + 1e-9) + b1_ref[...][None, :], 0.0)
    fex2_ref[...] = jnp.dot(h1, wa_ref[...], preferred_element_type=_f32)
    er2_ref[...] = jnp.dot(h1, wb_ref[...], preferred_element_type=_f32)


_mid = pl.pallas_call(
    _mid_body,
    grid=(N // RB,),
    in_specs=[
        pl.BlockSpec((RB, C1), lambda i: (i, 0)),
        pl.BlockSpec((RB, C1), lambda i: (i, 0)),
        pl.BlockSpec((8, HD), lambda i: (0, 0)),
        pl.BlockSpec((HD,), lambda i: (0,)),
        pl.BlockSpec((HD, C2), lambda i: (0, 0)),
        pl.BlockSpec((HD, 16), lambda i: (0, 0)),
    ],
    out_specs=[
        pl.BlockSpec((RB, C2), lambda i: (i, 0)),
        pl.BlockSpec((RB, 16), lambda i: (i, 0)),
    ],
    out_shape=[
        jax.ShapeDtypeStruct((NPAD, C2), _f32),
        jax.ShapeDtypeStruct((NPAD, 16), _f32),
    ],
)


def _final_body(q0_ref, q1_ref, b2_ref, o_ref):
    acc = q0_ref[...] + q1_ref[...]
    s2 = acc[:, OUT_DIM:OUT_DIM + 1]
    o_ref[...] = acc[:, :OUT_DIM] / (s2 + 1e-9) + b2_ref[...][None, :]


_final = pl.pallas_call(
    _final_body,
    grid=(N // RB,),
    in_specs=[
        pl.BlockSpec((RB, C2), lambda i: (i, 0)),
        pl.BlockSpec((RB, C2), lambda i: (i, 0)),
        pl.BlockSpec((OUT_DIM,), lambda i: (0,)),
    ],
    out_specs=pl.BlockSpec((RB, OUT_DIM), lambda i: (i, 0)),
    out_shape=jax.ShapeDtypeStruct((N, OUT_DIM), _f32),
)


def _edge_body(ncols, nheads, dim, fex_hbm, er_hbm, sd_hbm,
               zero_hbm, out_hbm, sdidx, cin, cout, erchunk, acc,
               semg, seme, sems, semi):
    c = lax.axis_index("c")
    s = lax.axis_index("s")
    w = s * NC + c
    iota = lax.iota(_i32, 16)
    lconsts = [jnp.full((16,), l, _i32) for l in range(16)]

    def idx_copy(i):
        islot = lax.rem(i, IDXN)
        pltpu.async_copy(sd_hbm.at[0, w, i], sdidx.at[islot, 0],
                         semi.at[islot])
        return pltpu.async_copy(sd_hbm.at[1, w, i], sdidx.at[islot, 1],
                                semi.at[islot])

    def start_gathers(i, slot):
        islot = lax.rem(i, IDXN)
        pltpu.async_copy(fex_hbm.at[sdidx.at[islot, 0]], cin.at[slot],
                         semg.at[slot])
        pltpu.async_copy(er_hbm.at[sdidx.at[islot, 1]], erchunk.at[slot],
                         seme.at[slot])

    def idx_wait(i):
        islot = lax.rem(i, IDXN)
        pltpu.make_async_copy(sd_hbm.at[0, w, i], sdidx.at[islot, 0],
                              semi.at[islot]).wait()
        pltpu.make_async_copy(sd_hbm.at[1, w, i], sdidx.at[islot, 1],
                              semi.at[islot]).wait()

    idx_copy(0)
    idx_wait(0)
    idx_copy(1)
    start_gathers(0, 0)

    @pl.when(s < NS - 1)
    def _():
        pltpu.sync_copy(zero_hbm.at[pl.ds(s * ZSH, ZSH)],
                        acc.at[pl.ds(s * ZSH, ZSH)])

    @pl.when(s == NS - 1)
    def _():
        r0 = (NS - 1) * ZSH
        pltpu.sync_copy(zero_hbm.at[pl.ds(r0, NPAD - r0)],
                        acc.at[pl.ds(r0, NPAD - r0)])

    plsc.subcore_barrier()

    nvec = dim // 16
    wcol = nheads * dim

    eu = max(1, 16 // (nheads * nvec))

    def step(i, slot):
        prev = 1 - slot
        islot = lax.rem(i, IDXN)
        islot_n = lax.rem(i + 1, IDXN)
        slotv = lconsts[slot]

        @pl.when(i >= NBUF)
        def _():
            pltpu.make_async_copy(
                cout.at[slot], acc.at[sdidx.at[lax.rem(i + 2, IDXN), 1]],
                sems.at[slot]).wait()

        @pl.when(i + 2 < NIT)
        def _():
            idx_copy(i + 2)

        @pl.when(i + 1 < NIT)
        def _():
            idx_wait(i + 1)
            start_gathers(i + 1, prev)

        pltpu.make_async_copy(fex_hbm.at[sdidx.at[islot, 0]], cin.at[slot],
                              semg.at[slot]).wait()
        pltpu.make_async_copy(er_hbm.at[sdidx.at[islot, 1]],
                              erchunk.at[slot], seme.at[slot]).wait()

        for g in range(SUB // 16):
            rows = g * 16 + iota
            els = [plsc.load_gather(
                cin, [slotv, rows, jnp.full((16,), wcol + h, _i32)])
                for h in range(nheads)]
            ers = [plsc.load_gather(
                erchunk, [slotv, rows, jnp.full((16,), h, _i32)])
                for h in range(nheads)]
            ts = [els[h] + ers[h] for h in range(nheads)]
            ws = [jnp.exp(jnp.maximum(t, 0.2 * t)) for t in ts]
            for h in range(nheads):
                plsc.store_scatter(
                    cout, [slotv, rows, jnp.full((16,), wcol + h, _i32)],
                    ws[h])
            for l0 in range(0, 16, eu):
                vals = [cin[slot, g * 16 + l0 + u, pl.ds(k * 16, 16)]
                        for u in range(eu) for k in range(nheads * nvec)]
                sps = [_dyn_gather(ws[h], lconsts[l0 + u])
                       for u in range(eu) for h in range(nheads)]
                prods = [vals[u * nheads * nvec + h * nvec + v]
                         * sps[u * nheads + h]
                         for u in range(eu) for h in range(nheads)
                         for v in range(nvec)]
                for u in range(eu):
                    for k in range(nheads * nvec):
                        cout[slot, g * 16 + l0 + u, pl.ds(k * 16, 16)] = (
                            prods[u * nheads * nvec + k])

        pltpu.async_copy(cout.at[slot], acc.at[sdidx.at[islot, 1]],
                         sems.at[slot], add=True)

    def it(j, carry):
        step(2 * j, 0)
        step(2 * j + 1, 1)
        return carry

    lax.fori_loop(0, NIT // 2, it, 0)
    for k in range(NBUF):
        i_last = NIT - NBUF + k
        pltpu.make_async_copy(
            cout.at[i_last % NBUF], acc.at[sdidx.at[i_last % IDXN, 1]],
            sems.at[i_last % NBUF]).wait()
    plsc.subcore_barrier()

    @pl.when(s < NS - 1)
    def _():
        pltpu.sync_copy(acc.at[pl.ds(s * ZSH, ZSH)],
                        out_hbm.at[c, pl.ds(s * ZSH, ZSH)])

    @pl.when(s == NS - 1)
    def _():
        r0 = (NS - 1) * ZSH
        pltpu.sync_copy(acc.at[pl.ds(r0, N - r0)],
                        out_hbm.at[c, pl.ds(r0, N - r0)])


def _make_edge(ncols, nheads, dim):
    mesh = plsc.VectorSubcoreMesh(core_axis_name="c", subcore_axis_name="s",
                                  num_cores=NC, num_subcores=NS)
    return pl.kernel(
        functools.partial(_edge_body, ncols, nheads, dim),
        out_type=jax.ShapeDtypeStruct((NC, N, ncols), _f32),
        mesh=mesh,
        compiler_params=pltpu.CompilerParams(use_tc_tiling_on_sc=False,
                                             needs_layout_passes=False),
        scratch_types=[
            pltpu.VMEM((IDXN, 2, SUB), _i32),
            pltpu.VMEM((NBUF, SUB, ncols), _f32),
            pltpu.VMEM((NBUF, SUB, ncols), _f32),
            pltpu.VMEM((NBUF, SUB, 16), _f32),
            pltpu.VMEM_SHARED((NPAD, ncols), _f32),
            pltpu.SemaphoreType.DMA((NBUF,)),
            pltpu.SemaphoreType.DMA((NBUF,)),
            pltpu.SemaphoreType.DMA((NBUF,)),
            pltpu.SemaphoreType.DMA((IDXN,)),
        ],
    )


_edge1 = _make_edge(C1, HEADS, HID)
_edge2 = _make_edge(C2, 1, OUT_DIM)


def kernel(x, edge_index, W1, al1, ar1, b1, W2, al2, ar2, b2):
    ei = edge_index.astype(_i32)
    npad = E_PAD - ei.shape[1]
    padidx = N + (jnp.arange(npad, dtype=_i32) % 16)
    eip = jnp.concatenate([ei, jnp.broadcast_to(padidx, (2, npad))], axis=1)
    sd = eip.reshape(2, NW, NIT, SUB)

    E8 = (jnp.arange(HD)[None, :] // HID == jnp.arange(8)[:, None]).astype(_f32)
    Al = E8.T * al1.reshape(-1)[:, None]
    Ar = E8.T * ar1.reshape(-1)[:, None]
    z = jnp.zeros((IN_DIM, 8), _f32)
    Wb1a = jnp.concatenate([W1, W1 @ Al, z], axis=1)
    Wb1b = jnp.concatenate([W1 @ Ar, z], axis=1)
    z15 = jnp.zeros((HD, 15), _f32)
    Wb2a = jnp.concatenate([W2, W2 @ al2.T, z15], axis=1)
    Wb2b = jnp.concatenate([W2 @ ar2.T, z15], axis=1)

    fex1, er1 = _mm1(x, Wb1a, Wb1b)
    p = _edge1(fex1, er1, sd, jnp.zeros((NPAD, C1), _f32))

    fex2, er2 = _mid(p[0], p[1], E8, b1, Wb2a, Wb2b)
    q = _edge2(fex2, er2, sd, jnp.zeros((NPAD, C2), _f32))

    return _final(q[0], q[1], b2)


# speedup vs baseline: 1.0369x; 1.0369x over previous
import functools

import jax
import jax.numpy as jnp
from jax import lax
from jax.experimental import pallas as pl
from jax.experimental.pallas import tpu as pltpu
from jax.experimental.pallas import tpu_sc as plsc

N = 10000
IN_DIM = 128
HID = 16
HEADS = 8
OUT_DIM = 64
HD = HEADS * HID

NC = 2
NS = 16
NW = NC * NS
SUB = 64
NBUF = 2
IDXN = 4
E_RAW = 320000
NIT = -(-E_RAW // (NW * SUB))
NIT += NIT % 2
EPW = NIT * SUB
E_PAD = EPW * NW

NPAD = N + 16
ZSH = 632
C1 = HD + 16
C2 = 80
RB = 400

_f32 = jnp.float32
_i32 = jnp.int32

_GD = lax.GatherDimensionNumbers(offset_dims=(), collapsed_slice_dims=(0,),
                                 start_index_map=(0,))


def _dyn_gather(vec, idx):
    return lax.gather(vec, idx[:, None], _GD, (1,),
                      mode=lax.GatherScatterMode.PROMISE_IN_BOUNDS)


def _mm1_body(x_ref, wa_ref, wb_ref, fex_ref, er_ref):
    fex_ref[...] = jnp.dot(x_ref[...], wa_ref[...], preferred_element_type=_f32)
    er_ref[...] = jnp.dot(x_ref[...], wb_ref[...], preferred_element_type=_f32)


_mm1 = pl.pallas_call(
    _mm1_body,
    grid=(N // RB,),
    in_specs=[
        pl.BlockSpec((RB, IN_DIM), lambda i: (i, 0)),
        pl.BlockSpec((IN_DIM, C1), lambda i: (0, 0)),
        pl.BlockSpec((IN_DIM, 16), lambda i: (0, 0)),
    ],
    out_specs=[
        pl.BlockSpec((RB, C1), lambda i: (i, 0)),
        pl.BlockSpec((RB, 16), lambda i: (i, 0)),
    ],
    out_shape=[
        jax.ShapeDtypeStruct((NPAD, C1), _f32),
        jax.ShapeDtypeStruct((NPAD, 16), _f32),
    ],
)


def _mid_body(p0_ref, p1_ref, e8_ref, b1_ref, wa_ref, wb_ref,
              fex2_ref, er2_ref):
    acc = p0_ref[...] + p1_ref[...]
    s8 = acc[:, HD:HD + 8]
    sfull = jnp.dot(s8, e8_ref[...], preferred_element_type=_f32)
    h1 = jnp.maximum(acc[:, :HD] / (sfull + 1e-9) + b1_ref[...][None, :], 0.0)
    fex2_ref[...] = jnp.dot(h1, wa_ref[...], preferred_element_type=_f32)
    er2_ref[...] = jnp.dot(h1, wb_ref[...], preferred_element_type=_f32)


_mid = pl.pallas_call(
    _mid_body,
    grid=(N // RB,),
    in_specs=[
        pl.BlockSpec((RB, C1), lambda i: (i, 0)),
        pl.BlockSpec((RB, C1), lambda i: (i, 0)),
        pl.BlockSpec((8, HD), lambda i: (0, 0)),
        pl.BlockSpec((HD,), lambda i: (0,)),
        pl.BlockSpec((HD, C2), lambda i: (0, 0)),
        pl.BlockSpec((HD, 16), lambda i: (0, 0)),
    ],
    out_specs=[
        pl.BlockSpec((RB, C2), lambda i: (i, 0)),
        pl.BlockSpec((RB, 16), lambda i: (i, 0)),
    ],
    out_shape=[
        jax.ShapeDtypeStruct((NPAD, C2), _f32),
        jax.ShapeDtypeStruct((NPAD, 16), _f32),
    ],
)


def _final_body(q0_ref, q1_ref, b2_ref, o_ref):
    acc = q0_ref[...] + q1_ref[...]
    s2 = acc[:, OUT_DIM:OUT_DIM + 1]
    o_ref[...] = acc[:, :OUT_DIM] / (s2 + 1e-9) + b2_ref[...][None, :]


_final = pl.pallas_call(
    _final_body,
    grid=(N // RB,),
    in_specs=[
        pl.BlockSpec((RB, C2), lambda i: (i, 0)),
        pl.BlockSpec((RB, C2), lambda i: (i, 0)),
        pl.BlockSpec((OUT_DIM,), lambda i: (0,)),
    ],
    out_specs=pl.BlockSpec((RB, OUT_DIM), lambda i: (i, 0)),
    out_shape=jax.ShapeDtypeStruct((N, OUT_DIM), _f32),
)


def _edge_body(ncols, nheads, dim, fex_hbm, er_hbm, sd_hbm,
               zero_hbm, out_hbm, sdidx, cin, cout, erchunk, acc,
               semg, seme, sems, semi):
    c = lax.axis_index("c")
    s = lax.axis_index("s")
    w = s * NC + c
    iota = lax.iota(_i32, 16)
    lconsts = [jnp.full((16,), l, _i32) for l in range(16)]

    def idx_copy(i):
        islot = lax.rem(i, IDXN)
        return pltpu.async_copy(sd_hbm.at[w, i], sdidx.at[islot],
                                semi.at[islot])

    def start_gathers(i, slot):
        islot = lax.rem(i, IDXN)
        pltpu.async_copy(fex_hbm.at[sdidx.at[islot, 0]], cin.at[slot],
                         semg.at[slot])
        pltpu.async_copy(er_hbm.at[sdidx.at[islot, 1]], erchunk.at[slot],
                         seme.at[slot])

    def idx_wait(i):
        islot = lax.rem(i, IDXN)
        pltpu.make_async_copy(sd_hbm.at[w, i], sdidx.at[islot],
                              semi.at[islot]).wait()

    idx_copy(0)
    idx_wait(0)
    idx_copy(1)
    start_gathers(0, 0)

    @pl.when(s < NS - 1)
    def _():
        pltpu.sync_copy(zero_hbm.at[pl.ds(s * ZSH, ZSH)],
                        acc.at[pl.ds(s * ZSH, ZSH)])

    @pl.when(s == NS - 1)
    def _():
        r0 = (NS - 1) * ZSH
        pltpu.sync_copy(zero_hbm.at[pl.ds(r0, NPAD - r0)],
                        acc.at[pl.ds(r0, NPAD - r0)])

    plsc.subcore_barrier()

    nvec = dim // 16
    wcol = nheads * dim

    eu = max(1, 16 // (nheads * nvec))

    def step(i, slot):
        prev = 1 - slot
        islot = lax.rem(i, IDXN)
        islot_n = lax.rem(i + 1, IDXN)
        slotv = lconsts[slot]

        @pl.when(i >= NBUF)
        def _():
            pltpu.make_async_copy(
                cout.at[slot], acc.at[sdidx.at[lax.rem(i + 2, IDXN), 1]],
                sems.at[slot]).wait()

        @pl.when(i + 2 < NIT)
        def _():
            idx_copy(i + 2)

        @pl.when(i + 1 < NIT)
        def _():
            idx_wait(i + 1)
            start_gathers(i + 1, prev)

        pltpu.make_async_copy(fex_hbm.at[sdidx.at[islot, 0]], cin.at[slot],
                              semg.at[slot]).wait()
        pltpu.make_async_copy(er_hbm.at[sdidx.at[islot, 1]],
                              erchunk.at[slot], seme.at[slot]).wait()

        for g in range(SUB // 16):
            rows = g * 16 + iota
            els = [plsc.load_gather(
                cin, [slotv, rows, jnp.full((16,), wcol + h, _i32)])
                for h in range(nheads)]
            ers = [plsc.load_gather(
                erchunk, [slotv, rows, jnp.full((16,), h, _i32)])
                for h in range(nheads)]
            ts = [els[h] + ers[h] for h in range(nheads)]
            ws = [jnp.exp(jnp.maximum(t, 0.2 * t)) for t in ts]
            for h in range(nheads):
                plsc.store_scatter(
                    cout, [slotv, rows, jnp.full((16,), wcol + h, _i32)],
                    ws[h])
            for l0 in range(0, 16, eu):
                vals = [cin[slot, g * 16 + l0 + u, pl.ds(k * 16, 16)]
                        for u in range(eu) for k in range(nheads * nvec)]
                sps = [_dyn_gather(ws[h], lconsts[l0 + u])
                       for u in range(eu) for h in range(nheads)]
                prods = [vals[u * nheads * nvec + h * nvec + v]
                         * sps[u * nheads + h]
                         for u in range(eu) for h in range(nheads)
                         for v in range(nvec)]
                for u in range(eu):
                    for k in range(nheads * nvec):
                        cout[slot, g * 16 + l0 + u, pl.ds(k * 16, 16)] = (
                            prods[u * nheads * nvec + k])

        pltpu.async_copy(cout.at[slot], acc.at[sdidx.at[islot, 1]],
                         sems.at[slot], add=True)

    def it(j, carry):
        step(2 * j, 0)
        step(2 * j + 1, 1)
        return carry

    lax.fori_loop(0, NIT // 2, it, 0)
    for k in range(NBUF):
        i_last = NIT - NBUF + k
        pltpu.make_async_copy(
            cout.at[i_last % NBUF], acc.at[sdidx.at[i_last % IDXN, 1]],
            sems.at[i_last % NBUF]).wait()
    plsc.subcore_barrier()

    @pl.when(s < NS - 1)
    def _():
        pltpu.sync_copy(acc.at[pl.ds(s * ZSH, ZSH)],
                        out_hbm.at[c, pl.ds(s * ZSH, ZSH)])

    @pl.when(s == NS - 1)
    def _():
        r0 = (NS - 1) * ZSH
        pltpu.sync_copy(acc.at[pl.ds(r0, N - r0)],
                        out_hbm.at[c, pl.ds(r0, N - r0)])


def _make_edge(ncols, nheads, dim):
    mesh = plsc.VectorSubcoreMesh(core_axis_name="c", subcore_axis_name="s",
                                  num_cores=NC, num_subcores=NS)
    return pl.kernel(
        functools.partial(_edge_body, ncols, nheads, dim),
        out_type=jax.ShapeDtypeStruct((NC, N, ncols), _f32),
        mesh=mesh,
        compiler_params=pltpu.CompilerParams(use_tc_tiling_on_sc=False,
                                             needs_layout_passes=False),
        scratch_types=[
            pltpu.VMEM((IDXN, 2, SUB), _i32),
            pltpu.VMEM((NBUF, SUB, ncols), _f32),
            pltpu.VMEM((NBUF, SUB, ncols), _f32),
            pltpu.VMEM((NBUF, SUB, 16), _f32),
            pltpu.VMEM_SHARED((NPAD, ncols), _f32),
            pltpu.SemaphoreType.DMA((NBUF,)),
            pltpu.SemaphoreType.DMA((NBUF,)),
            pltpu.SemaphoreType.DMA((NBUF,)),
            pltpu.SemaphoreType.DMA((IDXN,)),
        ],
    )


_edge1 = _make_edge(C1, HEADS, HID)
_edge2 = _make_edge(C2, 1, OUT_DIM)


def kernel(x, edge_index, W1, al1, ar1, b1, W2, al2, ar2, b2):
    ei = edge_index.astype(_i32)
    npad = E_PAD - ei.shape[1]
    padidx = N + (jnp.arange(npad, dtype=_i32) % 16)
    eip = jnp.concatenate([ei, jnp.broadcast_to(padidx, (2, npad))], axis=1)
    sd = eip.reshape(2, NW, NIT, SUB).transpose(1, 2, 0, 3)

    E8 = (jnp.arange(HD)[None, :] // HID == jnp.arange(8)[:, None]).astype(_f32)
    Al = E8.T * al1.reshape(-1)[:, None]
    Ar = E8.T * ar1.reshape(-1)[:, None]
    z = jnp.zeros((IN_DIM, 8), _f32)
    Wb1a = jnp.concatenate([W1, W1 @ Al, z], axis=1)
    Wb1b = jnp.concatenate([W1 @ Ar, z], axis=1)
    z15 = jnp.zeros((HD, 15), _f32)
    Wb2a = jnp.concatenate([W2, W2 @ al2.T, z15], axis=1)
    Wb2b = jnp.concatenate([W2 @ ar2.T, z15], axis=1)

    fex1, er1 = _mm1(x, Wb1a, Wb1b)
    p = _edge1(fex1, er1, sd, jnp.zeros((NPAD, C1), _f32))

    fex2, er2 = _mid(p[0], p[1], E8, b1, Wb2a, Wb2b)
    q = _edge2(fex2, er2, sd, jnp.zeros((NPAD, C2), _f32))

    return _final(q[0], q[1], b2)


# speedup vs baseline: 1.1105x; 1.0709x over previous
import functools

import jax
import jax.numpy as jnp
from jax import lax
from jax.experimental import pallas as pl
from jax.experimental.pallas import tpu as pltpu
from jax.experimental.pallas import tpu_sc as plsc

N = 10000
IN_DIM = 128
HID = 16
HEADS = 8
OUT_DIM = 64
HD = HEADS * HID

NC = 2
NS = 16
NW = NC * NS
SUB = 64
NBUF = 2
IDXN = 4
E_RAW = 320000
NIT = -(-E_RAW // (NW * SUB))
NIT += NIT % 2
EPW = NIT * SUB
E_PAD = EPW * NW

NPAD = N + 16
ZSH = 632
C1 = HD + 16
C2 = 80
RB = 2000

_f32 = jnp.float32
_i32 = jnp.int32

_GD = lax.GatherDimensionNumbers(offset_dims=(), collapsed_slice_dims=(0,),
                                 start_index_map=(0,))


def _dyn_gather(vec, idx):
    return lax.gather(vec, idx[:, None], _GD, (1,),
                      mode=lax.GatherScatterMode.PROMISE_IN_BOUNDS)


def _mm1_body(x_ref, wa_ref, wb_ref, fex_ref, er_ref):
    fex_ref[...] = jnp.dot(x_ref[...], wa_ref[...], preferred_element_type=_f32)
    er_ref[...] = jnp.dot(x_ref[...], wb_ref[...], preferred_element_type=_f32)


_mm1 = pl.pallas_call(
    _mm1_body,
    grid=(N // RB,),
    in_specs=[
        pl.BlockSpec((RB, IN_DIM), lambda i: (i, 0)),
        pl.BlockSpec((IN_DIM, C1), lambda i: (0, 0)),
        pl.BlockSpec((IN_DIM, 16), lambda i: (0, 0)),
    ],
    out_specs=[
        pl.BlockSpec((RB, C1), lambda i: (i, 0)),
        pl.BlockSpec((RB, 16), lambda i: (i, 0)),
    ],
    out_shape=[
        jax.ShapeDtypeStruct((NPAD, C1), _f32),
        jax.ShapeDtypeStruct((NPAD, 16), _f32),
    ],
)


def _mid_body(p0_ref, p1_ref, e8_ref, b1_ref, wa_ref, wb_ref,
              fex2_ref, er2_ref):
    acc = p0_ref[...] + p1_ref[...]
    s8 = acc[:, HD:HD + 8]
    sfull = jnp.dot(s8, e8_ref[...], preferred_element_type=_f32)
    h1 = jnp.maximum(acc[:, :HD] / (sfull + 1e-9) + b1_ref[...][None, :], 0.0)
    fex2_ref[...] = jnp.dot(h1, wa_ref[...], preferred_element_type=_f32)
    er2_ref[...] = jnp.dot(h1, wb_ref[...], preferred_element_type=_f32)


_mid = pl.pallas_call(
    _mid_body,
    grid=(N // RB,),
    in_specs=[
        pl.BlockSpec((RB, C1), lambda i: (i, 0)),
        pl.BlockSpec((RB, C1), lambda i: (i, 0)),
        pl.BlockSpec((8, HD), lambda i: (0, 0)),
        pl.BlockSpec((HD,), lambda i: (0,)),
        pl.BlockSpec((HD, C2), lambda i: (0, 0)),
        pl.BlockSpec((HD, 16), lambda i: (0, 0)),
    ],
    out_specs=[
        pl.BlockSpec((RB, C2), lambda i: (i, 0)),
        pl.BlockSpec((RB, 16), lambda i: (i, 0)),
    ],
    out_shape=[
        jax.ShapeDtypeStruct((NPAD, C2), _f32),
        jax.ShapeDtypeStruct((NPAD, 16), _f32),
    ],
)


def _final_body(q0_ref, q1_ref, b2_ref, o_ref):
    acc = q0_ref[...] + q1_ref[...]
    s2 = acc[:, OUT_DIM:OUT_DIM + 1]
    o_ref[...] = acc[:, :OUT_DIM] / (s2 + 1e-9) + b2_ref[...][None, :]


_final = pl.pallas_call(
    _final_body,
    grid=(N // RB,),
    in_specs=[
        pl.BlockSpec((RB, C2), lambda i: (i, 0)),
        pl.BlockSpec((RB, C2), lambda i: (i, 0)),
        pl.BlockSpec((OUT_DIM,), lambda i: (0,)),
    ],
    out_specs=pl.BlockSpec((RB, OUT_DIM), lambda i: (i, 0)),
    out_shape=jax.ShapeDtypeStruct((N, OUT_DIM), _f32),
)


def _edge_body(ncols, nheads, dim, fex_hbm, er_hbm, sd_hbm,
               zero_hbm, out_hbm, sdidx, cin, cout, erchunk, acc,
               semg, seme, sems, semi):
    c = lax.axis_index("c")
    s = lax.axis_index("s")
    w = s * NC + c
    iota = lax.iota(_i32, 16)
    lconsts = [jnp.full((16,), l, _i32) for l in range(16)]

    def idx_copy(i):
        islot = lax.rem(i, IDXN)
        return pltpu.async_copy(sd_hbm.at[w, i], sdidx.at[islot],
                                semi.at[islot])

    def start_gathers(i, slot):
        islot = lax.rem(i, IDXN)
        pltpu.async_copy(fex_hbm.at[sdidx.at[islot, 0]], cin.at[slot],
                         semg.at[slot])
        pltpu.async_copy(er_hbm.at[sdidx.at[islot, 1]], erchunk.at[slot],
                         seme.at[slot])

    def idx_wait(i):
        islot = lax.rem(i, IDXN)
        pltpu.make_async_copy(sd_hbm.at[w, i], sdidx.at[islot],
                              semi.at[islot]).wait()

    idx_copy(0)
    idx_wait(0)
    idx_copy(1)
    start_gathers(0, 0)

    @pl.when(s < NS - 1)
    def _():
        pltpu.sync_copy(zero_hbm.at[pl.ds(s * ZSH, ZSH)],
                        acc.at[pl.ds(s * ZSH, ZSH)])

    @pl.when(s == NS - 1)
    def _():
        r0 = (NS - 1) * ZSH
        pltpu.sync_copy(zero_hbm.at[pl.ds(r0, NPAD - r0)],
                        acc.at[pl.ds(r0, NPAD - r0)])

    plsc.subcore_barrier()

    nvec = dim // 16
    wcol = nheads * dim

    eu = max(1, 16 // (nheads * nvec))

    def step(i, slot):
        prev = 1 - slot
        islot = lax.rem(i, IDXN)
        islot_n = lax.rem(i + 1, IDXN)
        slotv = lconsts[slot]

        @pl.when(i >= NBUF)
        def _():
            pltpu.make_async_copy(
                cout.at[slot], acc.at[sdidx.at[lax.rem(i + 2, IDXN), 1]],
                sems.at[slot]).wait()

        @pl.when(i + 2 < NIT)
        def _():
            idx_copy(i + 2)

        @pl.when(i + 1 < NIT)
        def _():
            idx_wait(i + 1)
            start_gathers(i + 1, prev)

        pltpu.make_async_copy(fex_hbm.at[sdidx.at[islot, 0]], cin.at[slot],
                              semg.at[slot]).wait()
        pltpu.make_async_copy(er_hbm.at[sdidx.at[islot, 1]],
                              erchunk.at[slot], seme.at[slot]).wait()

        for g in range(SUB // 16):
            rows = g * 16 + iota
            els = [plsc.load_gather(
                cin, [slotv, rows, jnp.full((16,), wcol + h, _i32)])
                for h in range(nheads)]
            ers = [plsc.load_gather(
                erchunk, [slotv, rows, jnp.full((16,), h, _i32)])
                for h in range(nheads)]
            ts = [els[h] + ers[h] for h in range(nheads)]
            ws = [jnp.exp(jnp.maximum(t, 0.2 * t)) for t in ts]
            for h in range(nheads):
                plsc.store_scatter(
                    cout, [slotv, rows, jnp.full((16,), wcol + h, _i32)],
                    ws[h])
            for l0 in range(0, 16, eu):
                vals = [cin[slot, g * 16 + l0 + u, pl.ds(k * 16, 16)]
                        for u in range(eu) for k in range(nheads * nvec)]
                sps = [_dyn_gather(ws[h], lconsts[l0 + u])
                       for u in range(eu) for h in range(nheads)]
                prods = [vals[u * nheads * nvec + h * nvec + v]
                         * sps[u * nheads + h]
                         for u in range(eu) for h in range(nheads)
                         for v in range(nvec)]
                for u in range(eu):
                    for k in range(nheads * nvec):
                        cout[slot, g * 16 + l0 + u, pl.ds(k * 16, 16)] = (
                            prods[u * nheads * nvec + k])

        pltpu.async_copy(cout.at[slot], acc.at[sdidx.at[islot, 1]],
                         sems.at[slot], add=True)

    def it(j, carry):
        step(2 * j, 0)
        step(2 * j + 1, 1)
        return carry

    lax.fori_loop(0, NIT // 2, it, 0)
    for k in range(NBUF):
        i_last = NIT - NBUF + k
        pltpu.make_async_copy(
            cout.at[i_last % NBUF], acc.at[sdidx.at[i_last % IDXN, 1]],
            sems.at[i_last % NBUF]).wait()
    plsc.subcore_barrier()

    @pl.when(s < NS - 1)
    def _():
        pltpu.sync_copy(acc.at[pl.ds(s * ZSH, ZSH)],
                        out_hbm.at[c, pl.ds(s * ZSH, ZSH)])

    @pl.when(s == NS - 1)
    def _():
        r0 = (NS - 1) * ZSH
        pltpu.sync_copy(acc.at[pl.ds(r0, N - r0)],
                        out_hbm.at[c, pl.ds(r0, N - r0)])


def _make_edge(ncols, nheads, dim):
    mesh = plsc.VectorSubcoreMesh(core_axis_name="c", subcore_axis_name="s",
                                  num_cores=NC, num_subcores=NS)
    return pl.kernel(
        functools.partial(_edge_body, ncols, nheads, dim),
        out_type=jax.ShapeDtypeStruct((NC, N, ncols), _f32),
        mesh=mesh,
        compiler_params=pltpu.CompilerParams(use_tc_tiling_on_sc=False,
                                             needs_layout_passes=False),
        scratch_types=[
            pltpu.VMEM((IDXN, 2, SUB), _i32),
            pltpu.VMEM((NBUF, SUB, ncols), _f32),
            pltpu.VMEM((NBUF, SUB, ncols), _f32),
            pltpu.VMEM((NBUF, SUB, 16), _f32),
            pltpu.VMEM_SHARED((NPAD, ncols), _f32),
            pltpu.SemaphoreType.DMA((NBUF,)),
            pltpu.SemaphoreType.DMA((NBUF,)),
            pltpu.SemaphoreType.DMA((NBUF,)),
            pltpu.SemaphoreType.DMA((IDXN,)),
        ],
    )


_edge1 = _make_edge(C1, HEADS, HID)
_edge2 = _make_edge(C2, 1, OUT_DIM)


def kernel(x, edge_index, W1, al1, ar1, b1, W2, al2, ar2, b2):
    ei = edge_index.astype(_i32)
    npad = E_PAD - ei.shape[1]
    padidx = N + (jnp.arange(npad, dtype=_i32) % 16)
    eip = jnp.concatenate([ei, jnp.broadcast_to(padidx, (2, npad))], axis=1)
    sd = eip.reshape(2, NW, NIT, SUB).transpose(1, 2, 0, 3)

    E8 = (jnp.arange(HD)[None, :] // HID == jnp.arange(8)[:, None]).astype(_f32)
    Al = E8.T * al1.reshape(-1)[:, None]
    Ar = E8.T * ar1.reshape(-1)[:, None]
    z = jnp.zeros((IN_DIM, 8), _f32)
    Wb1a = jnp.concatenate([W1, W1 @ Al, z], axis=1)
    Wb1b = jnp.concatenate([W1 @ Ar, z], axis=1)
    z15 = jnp.zeros((HD, 15), _f32)
    Wb2a = jnp.concatenate([W2, W2 @ al2.T, z15], axis=1)
    Wb2b = jnp.concatenate([W2 @ ar2.T, z15], axis=1)

    fex1, er1 = _mm1(x, Wb1a, Wb1b)
    p = _edge1(fex1, er1, sd, jnp.zeros((NPAD, C1), _f32))

    fex2, er2 = _mid(p[0], p[1], E8, b1, Wb2a, Wb2b)
    q = _edge2(fex2, er2, sd, jnp.zeros((NPAD, C2), _f32))

    return _final(q[0], q[1], b2)
